# baseline (speedup 1.0000x reference)
import functools

import jax
import jax.numpy as jnp
from jax import lax
from jax.experimental import pallas as pl
from jax.experimental.pallas import tpu as pltpu
from jax.experimental.pallas import tpu_sc as plsc

N, E, D, DE, H, K = 10000, 160000, 128, 16, 256, 16
TE = 2000
TN = 2000
NC, NS = 2, 16
NW = NC * NS
EPW = E // NW
CH = 1000
NCHUNK = EPW // CH
NPT = N // NS


def _elu(z):
    return jnp.where(z > 0, z, jnp.exp(z) - 1.0)


def _xp_body(x_ref, win_ref, bin_ref, out_ref):
    out_ref[...] = _elu(
        jnp.dot(x_ref[...], win_ref[...], preferred_element_type=jnp.float32)
        + bin_ref[...])


def _edge_msg_body(ea_ref, gs_ref, gd_ref, w1a_ref, w1b_ref, b1_ref,
                   w2_ref, b2_ref, r_ref, s_ref, out_ref):
    gs = gs_ref[...]
    dpos = gs[:, 16:32] - gd_ref[...]
    z = (jnp.dot(ea_ref[...], w1a_ref[...], preferred_element_type=jnp.float32)
         + jnp.dot(dpos, w1b_ref[...], preferred_element_type=jnp.float32)
         + b1_ref[...])
    h1 = _elu(z)
    kern = jnp.dot(h1, w2_ref[...], preferred_element_type=jnp.float32) + b2_ref[...]
    xrep = jnp.dot(gs[:, 0:16], r_ref[...], preferred_element_type=jnp.float32)
    out_ref[...] = jnp.dot(kern * xrep, s_ref[...],
                           preferred_element_type=jnp.float32)


def _node_body(xp_ref, agg_ref, wroot_ref, broot_ref, wn1_ref, bn1_ref,
               wn2_ref, bn2_ref, hn_ref, np_ref):
    xp = xp_ref[...]
    agg = agg_ref[0] + agg_ref[1]
    hn = _elu(jnp.dot(xp, wroot_ref[...], preferred_element_type=jnp.float32)
              + broot_ref[...] + agg)
    t = _elu(jnp.dot(hn, wn1_ref[...], preferred_element_type=jnp.float32)
             + bn1_ref[...])
    hn_ref[...] = hn
    np_ref[...] = jnp.dot(t, wn2_ref[...], preferred_element_type=jnp.float32) \
        + bn2_ref[...]


def _edge_pred_body(hs_ref, hd_ref, we1a_ref, we1b_ref, be1_ref,
                    we2_ref, be2_ref, out_ref):
    t = _elu(jnp.dot(hs_ref[...], we1a_ref[...], preferred_element_type=jnp.float32)
             + jnp.dot(hd_ref[...], we1b_ref[...], preferred_element_type=jnp.float32)
             + be1_ref[...])
    out_ref[...] = jnp.dot(t, we2_ref[...], preferred_element_type=jnp.float32) \
        + be2_ref[...]


def _full(shape):
    return pl.BlockSpec(shape, lambda i: (0,) * len(shape))


_SC_MESH = plsc.VectorSubcoreMesh(core_axis_name="c", subcore_axis_name="s")
_SC_PARAMS = pltpu.CompilerParams(use_tc_tiling_on_sc=False)


def _make_gather2(wa, wb):
    def body(taba_hbm, tabb_hbm, idxa_hbm, idxb_hbm, outa_hbm, outb_hbm,
             ia_v, ib_v, ra_v, rb_v,
             sga0, sga1, sgb0, sgb1, swa0, swa1, swb0, swb1):
        wid = lax.axis_index("s") * NC + lax.axis_index("c")
        base0 = wid * EPW
        pltpu.sync_copy(idxa_hbm.at[pl.ds(base0, EPW)], ia_v)
        pltpu.sync_copy(idxb_hbm.at[pl.ds(base0, EPW)], ib_v)
        sga = (sga0, sga1)
        sgb = (sgb0, sgb1)
        swa = (swa0, swa1)
        swb = (swb0, swb1)
        ga = [None] * NCHUNK
        gb = [None] * NCHUNK
        wa_ = [None] * NCHUNK
        wb_ = [None] * NCHUNK
        for j in range(NCHUNK):
            b = j % 2
            if j >= 2:
                wa_[j - 2].wait()
                wb_[j - 2].wait()
            ga[j] = pltpu.async_copy(
                taba_hbm.at[ia_v.at[pl.ds(j * CH, CH)]], ra_v.at[b], sga[b])
            gb[j] = pltpu.async_copy(
                tabb_hbm.at[ib_v.at[pl.ds(j * CH, CH)]], rb_v.at[b], sgb[b])
            if j >= 1:
                p = (j - 1) % 2
                ga[j - 1].wait()
                gb[j - 1].wait()
                wa_[j - 1] = pltpu.async_copy(
                    ra_v.at[p], outa_hbm.at[pl.ds(base0 + (j - 1) * CH, CH)],
                    swa[p])
                wb_[j - 1] = pltpu.async_copy(
                    rb_v.at[p], outb_hbm.at[pl.ds(base0 + (j - 1) * CH, CH)],
                    swb[p])
        jl = NCHUNK - 1
        b = jl % 2
        ga[jl].wait()
        gb[jl].wait()
        wa_[jl] = pltpu.async_copy(
            ra_v.at[b], outa_hbm.at[pl.ds(base0 + jl * CH, CH)], swa[b])
        wb_[jl] = pltpu.async_copy(
            rb_v.at[b], outb_hbm.at[pl.ds(base0 + jl * CH, CH)], swb[b])
        wa_[jl - 1].wait()
        wb_[jl - 1].wait()
        wa_[jl].wait()
        wb_[jl].wait()

    return pl.kernel(
        body,
        out_type=[jax.ShapeDtypeStruct((E, wa), jnp.float32),
                  jax.ShapeDtypeStruct((E, wb), jnp.float32)],
        mesh=_SC_MESH,
        scratch_types=[pltpu.VMEM((EPW,), jnp.int32),
                       pltpu.VMEM((EPW,), jnp.int32),
                       pltpu.VMEM((2, CH, wa), jnp.float32),
                       pltpu.VMEM((2, CH, wb), jnp.float32)]
        + [pltpu.SemaphoreType.DMA] * 8,
        compiler_params=_SC_PARAMS,
    )


def _seg_sum_body(msg_hbm, idx_hbm, zeros_hbm, out_hbm, idx_v, msg_v, acc_sh,
                  sl0, sl1, ss0, ss1):
    cid = lax.axis_index("c")
    sid = lax.axis_index("s")
    wid = sid * NC + cid
    base0 = wid * EPW
    pltpu.sync_copy(zeros_hbm.at[pl.ds(sid * NPT, NPT)],
                    acc_sh.at[pl.ds(sid * NPT, NPT)])
    pltpu.sync_copy(idx_hbm.at[pl.ds(base0, EPW)], idx_v)
    plsc.subcore_barrier()
    sl = (sl0, sl1)
    ss = (ss0, ss1)
    ld = [None] * NCHUNK
    sc = [None] * NCHUNK
    for j in range(NCHUNK):
        b = j % 2
        if j >= 2:
            sc[j - 2].wait()
        ld[j] = pltpu.async_copy(
            msg_hbm.at[pl.ds(base0 + j * CH, CH)], msg_v.at[b], sl[b])
        if j >= 1:
            p = (j - 1) % 2
            ld[j - 1].wait()
            sc[j - 1] = pltpu.async_copy(
                msg_v.at[p], acc_sh.at[idx_v.at[pl.ds((j - 1) * CH, CH)]],
                ss[p], add=True)
    jl = NCHUNK - 1
    ld[jl].wait()
    sc[jl] = pltpu.async_copy(
        msg_v.at[jl % 2], acc_sh.at[idx_v.at[pl.ds(jl * CH, CH)]],
        ss[jl % 2], add=True)
    sc[jl - 1].wait()
    sc[jl].wait()
    plsc.subcore_barrier()
    pltpu.sync_copy(acc_sh.at[pl.ds(sid * NPT, NPT)],
                    out_hbm.at[cid, pl.ds(sid * NPT, NPT)])


_seg_sum = pl.kernel(
    _seg_sum_body,
    out_type=jax.ShapeDtypeStruct((NC, N, K), jnp.float32),
    mesh=_SC_MESH,
    scratch_types=[pltpu.VMEM((EPW,), jnp.int32),
                   pltpu.VMEM((2, CH, K), jnp.float32),
                   pltpu.VMEM_SHARED((N, K), jnp.float32)]
    + [pltpu.SemaphoreType.DMA] * 4,
    compiler_params=_SC_PARAMS,
)


def kernel(x, edge_index, edge_attr, batch, pos, W1, b1, W2, b2, Win, bin_,
           Wroot, broot, Wn1, bn1, Wn2, bn2, We1, be1, We2, be2):
    f32 = jnp.float32
    src = edge_index[0]
    dst = edge_index[1]

    xp = pl.pallas_call(
        _xp_body,
        grid=(N // TN,),
        in_specs=[pl.BlockSpec((TN, D), lambda i: (i, 0)),
                  _full((D, K)), _full((1, K))],
        out_specs=pl.BlockSpec((TN, K), lambda i: (i, 0)),
        out_shape=jax.ShapeDtypeStruct((N, K), f32),
    )(x, Win, bin_.reshape(1, K))

    pos_pad = jnp.pad(pos, ((0, 0), (0, 13)))
    tab = jnp.concatenate([xp, pos_pad], axis=1)
    gs, gd = _make_gather2(2 * K, K)(tab, pos_pad, src, dst)

    r_sel = (jnp.arange(H)[None, :] // K == jnp.arange(K)[:, None]).astype(f32)
    s_sel = (jnp.arange(H)[:, None] % K == jnp.arange(K)[None, :]).astype(f32)

    W1a = W1[:DE]
    W1b = jnp.pad(W1[DE:], ((0, 13), (0, 0)))

    msg = pl.pallas_call(
        _edge_msg_body,
        grid=(E // TE,),
        in_specs=[pl.BlockSpec((TE, DE), lambda i: (i, 0)),
                  pl.BlockSpec((TE, 2 * K), lambda i: (i, 0)),
                  pl.BlockSpec((TE, K), lambda i: (i, 0)),
                  _full((DE, H)), _full((K, H)), _full((1, H)),
                  _full((H, K * K)), _full((1, K * K)),
                  _full((K, H)), _full((H, K))],
        out_specs=pl.BlockSpec((TE, K), lambda i: (i, 0)),
        out_shape=jax.ShapeDtypeStruct((E, K), f32),
    )(edge_attr, gs, gd, W1a, W1b, b1.reshape(1, H), W2, b2.reshape(1, K * K),
      r_sel, s_sel)

    agg2 = _seg_sum(msg, dst, jnp.zeros((N, K), f32))

    hn, node_pred = pl.pallas_call(
        _node_body,
        grid=(N // TN,),
        in_specs=[pl.BlockSpec((TN, K), lambda i: (i, 0)),
                  pl.BlockSpec((NC, TN, K), lambda i: (0, i, 0)),
                  _full((K, K)), _full((1, K)),
                  _full((K, 64)), _full((1, 64)),
                  _full((64, 2)), _full((1, 2))],
        out_specs=[pl.BlockSpec((TN, K), lambda i: (i, 0)),
                   pl.BlockSpec((TN, 2), lambda i: (i, 0))],
        out_shape=[jax.ShapeDtypeStruct((N, K), f32),
                   jax.ShapeDtypeStruct((N, 2), f32)],
    )(xp, agg2, Wroot, broot.reshape(1, K), Wn1, bn1.reshape(1, 64),
      Wn2, bn2.reshape(1, 2))

    hs, hd = _make_gather2(K, K)(hn, hn, src, dst)
    edge_pred = pl.pallas_call(
        _edge_pred_body,
        grid=(E // TE,),
        in_specs=[pl.BlockSpec((TE, K), lambda i: (i, 0)),
                  pl.BlockSpec((TE, K), lambda i: (i, 0)),
                  _full((K, 64)), _full((K, 64)), _full((1, 64)),
                  _full((64, 2)), _full((1, 2))],
        out_specs=pl.BlockSpec((TE, 2), lambda i: (i, 0)),
        out_shape=jax.ShapeDtypeStruct((E, 2), f32),
    )(hs, hd, We1[:K], We1[K:], be1.reshape(1, 64), We2, be2.reshape(1, 2))

    return node_pred, edge_pred
